# baseline (speedup 1.0000x reference)
import jax
import jax.numpy as jnp
from jax.experimental import pallas as pl
from jax.experimental.pallas import tpu as pltpu

N = 10000
BM = 200
NUM_I = N // BM
P = 8
HIT_EVERY = 6


def _nan2num(v):
    return jnp.nan_to_num(v, nan=0.0, posinf=1.0, neginf=0.0)


def _hit_num(j):
    return jnp.minimum(j // HIT_EVERY, P)


def _is_hit(j):
    return (j > 0) & (j % HIT_EVERY == 0) & (j // HIT_EVERY <= P)


def _adj_index(s):
    j = s - NUM_I
    miss_block = j - _hit_num(j)
    pin_block = (j + 1) - _hit_num(j + 1)
    phase1 = jnp.where(_is_hit(j), pin_block, miss_block)
    return (jnp.where(s < NUM_I, s, phase1), 0)


def _out_index(s):
    j = s - NUM_I
    block = jnp.where(_is_hit(j), NUM_I - 1 - P + j // HIT_EVERY,
                      j - _hit_num(j))
    return (jnp.where(s < NUM_I, 0, block), 0)


def _gcn_body(x_ref, w1t_ref, b1_ref, w2t_ref, b2_ref, adj_ref, out_ref,
              a_scr, b_scr, cache_scr):
    s = pl.program_id(0)
    j = s - NUM_I
    hit = (s >= NUM_I) & _is_hit(j)

    @pl.when(s == 0)
    def _():
        xs = _nan2num(x_ref[...])
        a_scr[...] = (
            jnp.dot(xs, w1t_ref[...], preferred_element_type=jnp.float32)
            + b1_ref[...])

    @pl.when(s < NUM_I)
    def _():
        acc = jnp.dot(adj_ref[...], a_scr[...],
                      preferred_element_type=jnp.float32)
        h1 = jnp.maximum(_nan2num(acc), 0.0)
        b_scr[pl.ds(s * BM, BM), :] = (
            jnp.dot(h1, w2t_ref[...], preferred_element_type=jnp.float32)
            + b2_ref[...])

    @pl.when((s >= NUM_I - P) & (s < NUM_I))
    def _():
        cache_scr[s - (NUM_I - P)] = adj_ref[...].astype(jnp.bfloat16)

    @pl.when((s >= NUM_I) & jnp.logical_not(hit))
    def _():
        acc = jnp.dot(adj_ref[...], b_scr[...],
                      preferred_element_type=jnp.float32)
        out_ref[...] = _nan2num(acc)

    @pl.when(hit)
    def _():
        slot = j // HIT_EVERY - 1
        acc = jnp.dot(cache_scr[slot], b_scr[...].astype(jnp.bfloat16),
                      preferred_element_type=jnp.float32)
        out_ref[...] = _nan2num(acc)


@jax.jit
def kernel(x, init_adj, W1, b1, W2, b2):
    d_in = x.shape[1]
    d_hid = W1.shape[0]
    d_out = W2.shape[0]
    w1t = W1.T
    w2t = W2.T
    b1r = b1.reshape(1, d_hid)
    b2r = b2.reshape(1, d_out)

    out = pl.pallas_call(
        _gcn_body,
        grid=(2 * NUM_I,),
        in_specs=[
            pl.BlockSpec((N, d_in), lambda s: (0, 0)),
            pl.BlockSpec((d_in, d_hid), lambda s: (0, 0)),
            pl.BlockSpec((1, d_hid), lambda s: (0, 0)),
            pl.BlockSpec((d_hid, d_out), lambda s: (0, 0)),
            pl.BlockSpec((1, d_out), lambda s: (0, 0)),
            pl.BlockSpec((BM, N), _adj_index),
        ],
        out_specs=pl.BlockSpec((BM, d_out), _out_index),
        out_shape=jax.ShapeDtypeStruct((N, d_out), jnp.float32),
        scratch_shapes=[
            pltpu.VMEM((N, d_hid), jnp.float32),
            pltpu.VMEM((N, d_out), jnp.float32),
            pltpu.VMEM((P, BM, N), jnp.bfloat16),
        ],
        compiler_params=pltpu.CompilerParams(
            vmem_limit_bytes=64 * 1024 * 1024),
    )(x, w1t, b1r, w2t, b2r, init_adj)

    return out


# speedup vs baseline: 1.0528x; 1.0528x over previous
import jax
import jax.numpy as jnp
from jax.experimental import pallas as pl
from jax.experimental.pallas import tpu as pltpu

N = 10000
BM = 200
NUM_I = N // BM
P = 9
NUM1 = NUM_I - P


def _nan2num(v):
    return jnp.nan_to_num(v, nan=0.0, posinf=1.0, neginf=0.0)


def _gcn_body(x_ref, w1t_ref, b1_ref, w2t_ref, b2_ref, adj_ref,
              out1_ref, out2_ref, a_scr, b_scr, cache_scr):
    s = pl.program_id(0)

    @pl.when(s == 0)
    def _():
        xs = _nan2num(x_ref[...])
        a_scr[...] = (
            jnp.dot(xs, w1t_ref[...], preferred_element_type=jnp.float32)
            + b1_ref[...]).astype(jnp.bfloat16)

    @pl.when(s < NUM_I)
    def _():
        acc = jnp.dot(adj_ref[...], a_scr[...].astype(jnp.float32),
                      preferred_element_type=jnp.float32)
        h1 = jnp.maximum(_nan2num(acc), 0.0)
        b_scr[pl.ds(s * BM, BM), :] = (
            jnp.dot(h1, w2t_ref[...], preferred_element_type=jnp.float32)
            + b2_ref[...])

    @pl.when((s >= NUM_I - P) & (s < NUM_I))
    def _():
        cache_scr[s - (NUM_I - P)] = adj_ref[...].astype(jnp.bfloat16)

    @pl.when(s >= NUM_I)
    def _():
        acc = jnp.dot(adj_ref[...], b_scr[...],
                      preferred_element_type=jnp.float32)
        out1_ref[...] = _nan2num(acc)

    @pl.when((s >= NUM_I) & (s < NUM_I + P))
    def _():
        acc = jnp.dot(cache_scr[s - NUM_I], b_scr[...].astype(jnp.bfloat16),
                      preferred_element_type=jnp.float32)
        out2_ref[...] = _nan2num(acc)


@jax.jit
def kernel(x, init_adj, W1, b1, W2, b2):
    d_in = x.shape[1]
    d_hid = W1.shape[0]
    d_out = W2.shape[0]
    w1t = W1.T
    w2t = W2.T
    b1r = b1.reshape(1, d_hid)
    b2r = b2.reshape(1, d_out)

    out1, out2 = pl.pallas_call(
        _gcn_body,
        grid=(NUM_I + NUM1,),
        in_specs=[
            pl.BlockSpec((N, d_in), lambda s: (0, 0)),
            pl.BlockSpec((d_in, d_hid), lambda s: (0, 0)),
            pl.BlockSpec((1, d_hid), lambda s: (0, 0)),
            pl.BlockSpec((d_hid, d_out), lambda s: (0, 0)),
            pl.BlockSpec((1, d_out), lambda s: (0, 0)),
            pl.BlockSpec((BM, N),
                         lambda s: (jnp.where(s < NUM_I, s, s - NUM_I), 0)),
        ],
        out_specs=[
            pl.BlockSpec((BM, d_out),
                         lambda s: (jnp.where(s < NUM_I, 0, s - NUM_I), 0)),
            pl.BlockSpec((BM, d_out),
                         lambda s: (jnp.clip(s - NUM_I, 0, P - 1), 0)),
        ],
        out_shape=[
            jax.ShapeDtypeStruct((NUM1 * BM, d_out), jnp.float32),
            jax.ShapeDtypeStruct((P * BM, d_out), jnp.float32),
        ],
        scratch_shapes=[
            pltpu.VMEM((N, d_hid), jnp.bfloat16),
            pltpu.VMEM((N, d_out), jnp.float32),
            pltpu.VMEM((P, BM, N), jnp.bfloat16),
        ],
        compiler_params=pltpu.CompilerParams(
            vmem_limit_bytes=64 * 1024 * 1024),
    )(x, w1t, b1r, w2t, b2r, init_adj)

    return jnp.concatenate([out1, out2], axis=0)


# speedup vs baseline: 1.0640x; 1.0107x over previous
import jax
import jax.numpy as jnp
from jax.experimental import pallas as pl
from jax.experimental.pallas import tpu as pltpu

N = 10000
BM = 200
NUM_I = N // BM
P = 9
NUM1 = NUM_I - P


def _nan2num(v):
    return jnp.nan_to_num(v, nan=0.0, posinf=1.0, neginf=0.0)


def _adj_index(s):
    j = s - NUM_I
    return (jnp.where(s < NUM_I, s, jnp.where(j < P, 2 * j, j + P)), 0)


def _out_index(s):
    j = s - NUM_I
    return (jnp.where(s < NUM_I, 0, jnp.where(j < P, j, (j + P) // 2)), 0)


def _gcn_body(x_ref, w1t_ref, b1_ref, w2t_ref, b2_ref, adj_ref, out_ref,
              a_scr, b_scr, cache_scr):
    s = pl.program_id(0)
    j = s - NUM_I

    @pl.when(s == 0)
    def _():
        xs = _nan2num(x_ref[...])
        a_scr[...] = (
            jnp.dot(xs, w1t_ref[...], preferred_element_type=jnp.float32)
            + b1_ref[...]).astype(jnp.bfloat16)

    @pl.when(s < NUM_I)
    def _():
        acc = jnp.dot(adj_ref[...], a_scr[...].astype(jnp.float32),
                      preferred_element_type=jnp.float32)
        h1 = jnp.maximum(_nan2num(acc), 0.0)
        b_scr[pl.ds(s * BM, BM), :] = (
            jnp.dot(h1, w2t_ref[...], preferred_element_type=jnp.float32)
            + b2_ref[...]).astype(jnp.bfloat16)

    @pl.when((s < 2 * P) & (s % 2 == 1))
    def _():
        cache_scr[s // 2] = adj_ref[...].astype(jnp.bfloat16)

    @pl.when((s >= NUM_I) & (j < P))
    def _():
        acc = jnp.dot(adj_ref[...], b_scr[...].astype(jnp.float32),
                      preferred_element_type=jnp.float32)
        out_ref[pl.ds(0, BM), :] = _nan2num(acc)
        acc2 = jnp.dot(cache_scr[j], b_scr[...],
                       preferred_element_type=jnp.float32)
        out_ref[pl.ds(BM, BM), :] = _nan2num(acc2)

    @pl.when(j >= P)
    def _():
        r = j + P
        acc = jnp.dot(adj_ref[...], b_scr[...].astype(jnp.float32),
                      preferred_element_type=jnp.float32)
        out_ref[pl.ds((r % 2) * BM, BM), :] = _nan2num(acc)


@jax.jit
def kernel(x, init_adj, W1, b1, W2, b2):
    d_in = x.shape[1]
    d_hid = W1.shape[0]
    d_out = W2.shape[0]
    w1t = W1.T
    w2t = W2.T
    b1r = b1.reshape(1, d_hid)
    b2r = b2.reshape(1, d_out)

    out = pl.pallas_call(
        _gcn_body,
        grid=(NUM_I + NUM1,),
        in_specs=[
            pl.BlockSpec((N, d_in), lambda s: (0, 0)),
            pl.BlockSpec((d_in, d_hid), lambda s: (0, 0)),
            pl.BlockSpec((1, d_hid), lambda s: (0, 0)),
            pl.BlockSpec((d_hid, d_out), lambda s: (0, 0)),
            pl.BlockSpec((1, d_out), lambda s: (0, 0)),
            pl.BlockSpec((BM, N), _adj_index),
        ],
        out_specs=pl.BlockSpec((2 * BM, d_out), _out_index),
        out_shape=jax.ShapeDtypeStruct((N, d_out), jnp.float32),
        scratch_shapes=[
            pltpu.VMEM((N, d_hid), jnp.bfloat16),
            pltpu.VMEM((N, d_out), jnp.bfloat16),
            pltpu.VMEM((P, BM, N), jnp.bfloat16),
        ],
        compiler_params=pltpu.CompilerParams(
            vmem_limit_bytes=64 * 1024 * 1024),
    )(x, w1t, b1r, w2t, b2r, init_adj)

    return out
